# baseline (speedup 1.0000x reference)
import functools

import jax
import jax.numpy as jnp
from jax import lax
from jax.experimental import pallas as pl
from jax.experimental.pallas import tpu as pltpu
from jax.experimental.pallas import tpu_sc as plsc

N = 10000
E = 320000
D = 128

NC = 2
NS = 16
EPT = E // NS
C = 80
EPT_PAD = 20480
NCHUNK = EPT_PAD // C
N_PAD = 10240
RPT = N_PAD // NS
CHB = 16
NBLK = NCHUNK // CHB
NRING = 4


def _matmul(x, W0, W1):
    def body(x_ref, w0_ref, w1_ref, h0_ref, h1_ref):
        xb = x_ref[...]
        h0_ref[...] = jnp.dot(xb, w0_ref[...], preferred_element_type=jnp.float32)
        h1_ref[...] = jnp.dot(xb, w1_ref[...], preferred_element_type=jnp.float32)

    return pl.pallas_call(
        body,
        grid=(10,),
        in_specs=[
            pl.BlockSpec((N // 10, D), lambda i: (i, 0)),
            pl.BlockSpec((D, D), lambda i: (0, 0)),
            pl.BlockSpec((D, D), lambda i: (0, 0)),
        ],
        out_specs=[
            pl.BlockSpec((N // 10, D), lambda i: (i, 0)),
            pl.BlockSpec((N // 10, D), lambda i: (i, 0)),
        ],
        out_shape=[jax.ShapeDtypeStruct((N, D), jnp.float32)] * 2,
    )(x, W0, W1)


def _combine(p0, p1):
    def body(a_ref, b_ref, o_ref):
        o_ref[...] = jnp.maximum(a_ref[...] + b_ref[...], 0.0)

    return pl.pallas_call(
        body,
        grid=(10,),
        in_specs=[
            pl.BlockSpec((N // 10, D), lambda i: (i, 0)),
            pl.BlockSpec((N // 10, D), lambda i: (i, 0)),
        ],
        out_specs=pl.BlockSpec((N // 10, D), lambda i: (i, 0)),
        out_shape=jax.ShapeDtypeStruct((N, D), jnp.float32),
    )(p0, p1)


@functools.partial(
    pl.kernel,
    out_type=[
        jax.ShapeDtypeStruct((N_PAD, D), jnp.float32),
        jax.ShapeDtypeStruct((N_PAD, D), jnp.float32),
    ],
    mesh=plsc.VectorSubcoreMesh(core_axis_name="c", subcore_axis_name="s"),
    scratch_types=[
        pltpu.VMEM((CHB, C), jnp.int32),
        pltpu.VMEM((CHB, C), jnp.int32),
        pltpu.VMEM((CHB, C), jnp.float32),
        [pltpu.VMEM((C, D), jnp.float32)] * NRING,
        pltpu.VMEM_SHARED((N_PAD, D), jnp.float32),
        [pltpu.SemaphoreType.DMA] * NRING,
        [pltpu.SemaphoreType.DMA] * NRING,
    ],
)
def _spmm_sc(h0, h1, r0, c0, v0, r1, c1, v1, p0, p1,
             rowv, colv, valv, rbufs, accum, gsems, ssems):
    c = lax.axis_index("c")
    s = lax.axis_index("s")

    def zfill(r, _):
        for f in range(D // 16):
            rbufs[0][r, pl.ds(f * 16, 16)] = jnp.zeros((16,), jnp.float32)
        return 0

    lax.fori_loop(0, C, zfill, 0)

    def zcopy(j, _):
        pltpu.sync_copy(rbufs[0], accum.at[pl.ds(s * RPT + j * C, C)])
        return 0

    lax.fori_loop(0, RPT // C, zcopy, 0)
    plsc.subcore_barrier()

    def scale(buf, k):
        def group(g, _):
            v16 = valv[k, pl.ds(g * 16, 16)]
            ebase = g * 16
            for j in range(16):
                sv = v16[j]
                for f in range(D // 16):
                    sl = (ebase + j, pl.ds(f * 16, 16))
                    buf[sl] = buf[sl] * sv
            return 0

        lax.fori_loop(0, C // 16, group, 0)

    def run_support(h, r, cc, v):
        def blk(b, _):
            pltpu.sync_copy(r.at[s, b], rowv)
            pltpu.sync_copy(cc.at[s, b], colv)
            pltpu.sync_copy(v.at[s, b], valv)
            pltpu.async_copy(h.at[colv.at[0]], rbufs[0], gsems[0])
            pltpu.async_copy(h.at[colv.at[1]], rbufs[1], gsems[1])

            def quad(q, _):
                for i in range(NRING):
                    k = NRING * q + i
                    bf = i
                    nb = (i + 2) % NRING
                    pltpu.make_async_copy(
                        h.at[colv.at[k]], rbufs[bf], gsems[bf]).wait()
                    scale(rbufs[bf], k)
                    pltpu.async_copy(
                        rbufs[bf], accum.at[rowv.at[k]], ssems[bf], add=True)

                    @pl.when(k >= 2)
                    def _():
                        pltpu.make_async_copy(
                            rbufs[nb], accum.at[rowv.at[k - 2]],
                            ssems[nb]).wait()

                    @pl.when(k + 2 < CHB)
                    def _():
                        pltpu.async_copy(
                            h.at[colv.at[k + 2]], rbufs[nb], gsems[nb])

                return 0

            lax.fori_loop(0, CHB // NRING, quad, 0)
            for k in range(CHB - 2, CHB):
                pltpu.make_async_copy(
                    rbufs[k % NRING], accum.at[rowv.at[k]],
                    ssems[k % NRING]).wait()
            return 0

        lax.fori_loop(0, NBLK, blk, 0)

    @pl.when(c == 0)
    def _():
        run_support(h0, r0, c0, v0)

    @pl.when(c == 1)
    def _():
        run_support(h1, r1, c1, v1)

    plsc.subcore_barrier()

    @pl.when(c == 0)
    def _():
        pltpu.sync_copy(accum.at[pl.ds(s * RPT, RPT)], p0.at[pl.ds(s * RPT, RPT)])

    @pl.when(c == 1)
    def _():
        pltpu.sync_copy(accum.at[pl.ds(s * RPT, RPT)], p1.at[pl.ds(s * RPT, RPT)])


def kernel(x, support0_idx, support0_val, support1_idx, support1_val, W0, W1):
    h0, h1 = _matmul(x.astype(jnp.float32), W0, W1)

    pad = EPT_PAD - EPT
    pad_rows = jnp.broadcast_to(
        N + (jnp.arange(pad, dtype=jnp.int32) % (N_PAD - N)), (NS, pad))
    pad_cols = jnp.broadcast_to(
        (jnp.arange(pad, dtype=jnp.int32) * 37) % N, (NS, pad))

    def prep(a, padv):
        a = jnp.concatenate([a.reshape(NS, EPT), padv.astype(a.dtype)], axis=1)
        return a.reshape(NS, NBLK, CHB, C)

    zpad = jnp.zeros((NS, pad), jnp.float32)
    r0 = prep(support0_idx[0], pad_rows)
    c0 = prep(support0_idx[1], pad_cols)
    v0 = prep(support0_val, zpad)
    r1 = prep(support1_idx[0], pad_rows)
    c1 = prep(support1_idx[1], pad_cols)
    v1 = prep(support1_val, zpad)

    p0, p1 = _spmm_sc(h0, h1, r0, c0, v0, r1, c1, v1)
    return _combine(p0, p1)


# speedup vs baseline: 1.0130x; 1.0130x over previous
import functools

import jax
import jax.numpy as jnp
from jax import lax
from jax.experimental import pallas as pl
from jax.experimental.pallas import tpu as pltpu
from jax.experimental.pallas import tpu_sc as plsc

N = 10000
E = 320000
D = 128

NC = 2
NS = 16
EPT = E // NS
C = 128
EPT_PAD = 20480
NCHUNK = EPT_PAD // C
N_PAD = 10240
RPT = N_PAD // NS
CHB = 20
NBLK = NCHUNK // CHB
NRING = 2


def _matmul(x, W0, W1):
    def body(x_ref, w0_ref, w1_ref, h0_ref, h1_ref):
        xb = x_ref[...]
        h0_ref[...] = jnp.dot(xb, w0_ref[...], preferred_element_type=jnp.float32)
        h1_ref[...] = jnp.dot(xb, w1_ref[...], preferred_element_type=jnp.float32)

    return pl.pallas_call(
        body,
        grid=(10,),
        in_specs=[
            pl.BlockSpec((N // 10, D), lambda i: (i, 0)),
            pl.BlockSpec((D, D), lambda i: (0, 0)),
            pl.BlockSpec((D, D), lambda i: (0, 0)),
        ],
        out_specs=[
            pl.BlockSpec((N // 10, D), lambda i: (i, 0)),
            pl.BlockSpec((N // 10, D), lambda i: (i, 0)),
        ],
        out_shape=[jax.ShapeDtypeStruct((N, D), jnp.float32)] * 2,
    )(x, W0, W1)


def _combine(p0, p1):
    def body(a_ref, b_ref, o_ref):
        o_ref[...] = jnp.maximum(a_ref[...] + b_ref[...], 0.0)

    return pl.pallas_call(
        body,
        grid=(10,),
        in_specs=[
            pl.BlockSpec((N // 10, D), lambda i: (i, 0)),
            pl.BlockSpec((N // 10, D), lambda i: (i, 0)),
        ],
        out_specs=pl.BlockSpec((N // 10, D), lambda i: (i, 0)),
        out_shape=jax.ShapeDtypeStruct((N, D), jnp.float32),
    )(p0, p1)


@functools.partial(
    pl.kernel,
    out_type=[
        jax.ShapeDtypeStruct((N_PAD, D), jnp.float32),
        jax.ShapeDtypeStruct((N_PAD, D), jnp.float32),
    ],
    mesh=plsc.VectorSubcoreMesh(core_axis_name="c", subcore_axis_name="s"),
    scratch_types=[
        pltpu.VMEM((CHB, C), jnp.int32),
        pltpu.VMEM((CHB, C), jnp.int32),
        pltpu.VMEM((CHB, C), jnp.float32),
        [pltpu.VMEM((C, D), jnp.float32)] * NRING,
        pltpu.VMEM_SHARED((N_PAD, D), jnp.float32),
        [pltpu.SemaphoreType.DMA] * NRING,
        [pltpu.SemaphoreType.DMA] * NRING,
    ],
)
def _spmm_sc(h0, h1, r0, c0, v0, r1, c1, v1, p0, p1,
             rowv, colv, valv, rbufs, accum, gsems, ssems):
    c = lax.axis_index("c")
    s = lax.axis_index("s")

    def zfill(r, _):
        for f in range(D // 16):
            rbufs[0][r, pl.ds(f * 16, 16)] = jnp.zeros((16,), jnp.float32)
        return 0

    lax.fori_loop(0, C, zfill, 0)

    def zcopy(j, _):
        pltpu.sync_copy(rbufs[0], accum.at[pl.ds(s * RPT + j * C, C)])
        return 0

    lax.fori_loop(0, RPT // C, zcopy, 0)
    plsc.subcore_barrier()

    def scale(buf, k):
        def group(g, _):
            v16 = valv[k, pl.ds(g * 16, 16)]
            ebase = g * 16
            for j in range(16):
                sv = v16[j]
                for f in range(D // 16):
                    sl = (ebase + j, pl.ds(f * 16, 16))
                    buf[sl] = buf[sl] * sv
            return 0

        lax.fori_loop(0, C // 16, group, 0)

    def run_support(h, r, cc, v):
        def blk(b, _):
            pltpu.sync_copy(r.at[s, b], rowv)
            pltpu.sync_copy(cc.at[s, b], colv)
            pltpu.sync_copy(v.at[s, b], valv)
            pltpu.async_copy(h.at[colv.at[0]], rbufs[0], gsems[0])

            def duo(q, _):
                for i in range(NRING):
                    k = NRING * q + i
                    bf = i
                    nb = (i + 1) % NRING
                    @pl.when(k >= 1)
                    def _():
                        pltpu.make_async_copy(
                            rbufs[nb], accum.at[rowv.at[k - 1]],
                            ssems[nb]).wait()

                    @pl.when(k + 1 < CHB)
                    def _():
                        pltpu.async_copy(
                            h.at[colv.at[k + 1]], rbufs[nb], gsems[nb])

                    pltpu.make_async_copy(
                        h.at[colv.at[k]], rbufs[bf], gsems[bf]).wait()
                    scale(rbufs[bf], k)
                    pltpu.async_copy(
                        rbufs[bf], accum.at[rowv.at[k]], ssems[bf], add=True)

                return 0

            lax.fori_loop(0, CHB // NRING, duo, 0)
            pltpu.make_async_copy(
                rbufs[(CHB - 1) % NRING], accum.at[rowv.at[CHB - 1]],
                ssems[(CHB - 1) % NRING]).wait()
            return 0

        lax.fori_loop(0, NBLK, blk, 0)

    @pl.when(c == 0)
    def _():
        run_support(h0, r0, c0, v0)

    @pl.when(c == 1)
    def _():
        run_support(h1, r1, c1, v1)

    plsc.subcore_barrier()

    @pl.when(c == 0)
    def _():
        pltpu.sync_copy(accum.at[pl.ds(s * RPT, RPT)], p0.at[pl.ds(s * RPT, RPT)])

    @pl.when(c == 1)
    def _():
        pltpu.sync_copy(accum.at[pl.ds(s * RPT, RPT)], p1.at[pl.ds(s * RPT, RPT)])


def kernel(x, support0_idx, support0_val, support1_idx, support1_val, W0, W1):
    h0, h1 = _matmul(x.astype(jnp.float32), W0, W1)

    pad = EPT_PAD - EPT
    pad_rows = jnp.broadcast_to(
        N + (jnp.arange(pad, dtype=jnp.int32) % (N_PAD - N)), (NS, pad))
    pad_cols = jnp.broadcast_to(
        (jnp.arange(pad, dtype=jnp.int32) * 37) % N, (NS, pad))

    def prep(a, padv):
        a = jnp.concatenate([a.reshape(NS, EPT), padv.astype(a.dtype)], axis=1)
        return a.reshape(NS, NBLK, CHB, C)

    zpad = jnp.zeros((NS, pad), jnp.float32)
    r0 = prep(support0_idx[0], pad_rows)
    c0 = prep(support0_idx[1], pad_cols)
    v0 = prep(support0_val, zpad)
    r1 = prep(support1_idx[0], pad_rows)
    c1 = prep(support1_idx[1], pad_cols)
    v1 = prep(support1_val, zpad)

    p0, p1 = _spmm_sc(h0, h1, r0, c0, v0, r1, c1, v1)
    return _combine(p0, p1)


# speedup vs baseline: 1.0280x; 1.0148x over previous
import functools

import jax
import jax.numpy as jnp
from jax import lax
from jax.experimental import pallas as pl
from jax.experimental.pallas import tpu as pltpu
from jax.experimental.pallas import tpu_sc as plsc

N = 10000
E = 320000
D = 128

NC = 2
NS = 16
EPT = E // NS
C = 128
EPT_PAD = 20480
NCHUNK = EPT_PAD // C
N_PAD = 10240
RPT = N_PAD // NS
CHB = 20
NBLK = NCHUNK // CHB
NRING = 2


def _matmul(x, W0, W1):
    def body(x_ref, w0_ref, w1_ref, h0_ref, h1_ref):
        xb = x_ref[...]
        h0_ref[...] = jnp.dot(xb, w0_ref[...], preferred_element_type=jnp.float32)
        h1_ref[...] = jnp.dot(xb, w1_ref[...], preferred_element_type=jnp.float32)

    G = 5
    return pl.pallas_call(
        body,
        grid=(G,),
        in_specs=[
            pl.BlockSpec((N // G, D), lambda i: (i, 0)),
            pl.BlockSpec((D, D), lambda i: (0, 0)),
            pl.BlockSpec((D, D), lambda i: (0, 0)),
        ],
        out_specs=[
            pl.BlockSpec((N // G, D), lambda i: (i, 0)),
            pl.BlockSpec((N // G, D), lambda i: (i, 0)),
        ],
        out_shape=[jax.ShapeDtypeStruct((N, D), jnp.float32)] * 2,
    )(x, W0, W1)


def _combine(p0, p1):
    def body(a_ref, b_ref, o_ref):
        o_ref[...] = jnp.maximum(a_ref[...] + b_ref[...], 0.0)

    G = 5
    return pl.pallas_call(
        body,
        grid=(G,),
        in_specs=[
            pl.BlockSpec((N // G, D), lambda i: (i, 0)),
            pl.BlockSpec((N // G, D), lambda i: (i, 0)),
        ],
        out_specs=pl.BlockSpec((N // G, D), lambda i: (i, 0)),
        out_shape=jax.ShapeDtypeStruct((N, D), jnp.float32),
    )(p0, p1)


@functools.partial(
    pl.kernel,
    out_type=[
        jax.ShapeDtypeStruct((N_PAD, D), jnp.float32),
        jax.ShapeDtypeStruct((N_PAD, D), jnp.float32),
    ],
    mesh=plsc.VectorSubcoreMesh(core_axis_name="c", subcore_axis_name="s"),
    scratch_types=[
        pltpu.VMEM((CHB, C), jnp.int32),
        pltpu.VMEM((CHB, C), jnp.int32),
        pltpu.VMEM((CHB, C), jnp.float32),
        [pltpu.VMEM((C, D), jnp.float32)] * NRING,
        pltpu.VMEM_SHARED((N_PAD, D), jnp.float32),
        [pltpu.SemaphoreType.DMA] * NRING,
        [pltpu.SemaphoreType.DMA] * NRING,
    ],
)
def _spmm_sc(h0, h1, r0, c0, v0, r1, c1, v1, p0, p1,
             rowv, colv, valv, rbufs, accum, gsems, ssems):
    c = lax.axis_index("c")
    s = lax.axis_index("s")

    def zfill(r, _):
        for f in range(D // 16):
            rbufs[0][r, pl.ds(f * 16, 16)] = jnp.zeros((16,), jnp.float32)
        return 0

    lax.fori_loop(0, C, zfill, 0)

    def zcopy(j, _):
        pltpu.sync_copy(rbufs[0], accum.at[pl.ds(s * RPT + j * C, C)])
        return 0

    lax.fori_loop(0, RPT // C, zcopy, 0)
    plsc.subcore_barrier()

    def scale(buf, k):
        def group(g, _):
            v16 = valv[k, pl.ds(g * 16, 16)]
            ebase = g * 16
            for j in range(16):
                sv = v16[j]
                for f in range(D // 16):
                    sl = (ebase + j, pl.ds(f * 16, 16))
                    buf[sl] = buf[sl] * sv
            return 0

        lax.fori_loop(0, C // 16, group, 0)

    def run_support(h, r, cc, v):
        def blk(b, _):
            pltpu.sync_copy(r.at[s, b], rowv)
            pltpu.sync_copy(cc.at[s, b], colv)
            pltpu.sync_copy(v.at[s, b], valv)
            pltpu.async_copy(h.at[colv.at[0]], rbufs[0], gsems[0])

            def duo(q, _):
                for i in range(NRING):
                    k = NRING * q + i
                    bf = i
                    nb = (i + 1) % NRING
                    @pl.when(k >= 1)
                    def _():
                        pltpu.make_async_copy(
                            rbufs[nb], accum.at[rowv.at[k - 1]],
                            ssems[nb]).wait()

                    @pl.when(k + 1 < CHB)
                    def _():
                        pltpu.async_copy(
                            h.at[colv.at[k + 1]], rbufs[nb], gsems[nb])

                    pltpu.make_async_copy(
                        h.at[colv.at[k]], rbufs[bf], gsems[bf]).wait()
                    scale(rbufs[bf], k)
                    pltpu.async_copy(
                        rbufs[bf], accum.at[rowv.at[k]], ssems[bf], add=True)

                return 0

            lax.fori_loop(0, CHB // NRING, duo, 0)
            pltpu.make_async_copy(
                rbufs[(CHB - 1) % NRING], accum.at[rowv.at[CHB - 1]],
                ssems[(CHB - 1) % NRING]).wait()
            return 0

        lax.fori_loop(0, NBLK, blk, 0)

    @pl.when(c == 0)
    def _():
        run_support(h0, r0, c0, v0)

    @pl.when(c == 1)
    def _():
        run_support(h1, r1, c1, v1)

    plsc.subcore_barrier()

    @pl.when(c == 0)
    def _():
        pltpu.sync_copy(accum.at[pl.ds(s * RPT, RPT)], p0.at[pl.ds(s * RPT, RPT)])

    @pl.when(c == 1)
    def _():
        pltpu.sync_copy(accum.at[pl.ds(s * RPT, RPT)], p1.at[pl.ds(s * RPT, RPT)])


def kernel(x, support0_idx, support0_val, support1_idx, support1_val, W0, W1):
    h0, h1 = _matmul(x.astype(jnp.float32), W0, W1)

    pad = EPT_PAD - EPT
    pad_rows = jnp.broadcast_to(
        N + (jnp.arange(pad, dtype=jnp.int32) % (N_PAD - N)), (NS, pad))
    pad_cols = jnp.broadcast_to(
        (jnp.arange(pad, dtype=jnp.int32) * 37) % N, (NS, pad))

    def prep(a, padv):
        a = jnp.concatenate([a.reshape(NS, EPT), padv.astype(a.dtype)], axis=1)
        return a.reshape(NS, NBLK, CHB, C)

    zpad = jnp.zeros((NS, pad), jnp.float32)
    r0 = prep(support0_idx[0], pad_rows)
    c0 = prep(support0_idx[1], pad_cols)
    v0 = prep(support0_val, zpad)
    r1 = prep(support1_idx[0], pad_rows)
    c1 = prep(support1_idx[1], pad_cols)
    v1 = prep(support1_val, zpad)

    p0, p1 = _spmm_sc(h0, h1, r0, c0, v0, r1, c1, v1)
    return _combine(p0, p1)
